# baseline (speedup 1.0000x reference)
import functools

import jax
import jax.numpy as jnp
from jax import lax
from jax.experimental import pallas as pl
from jax.experimental.pallas import tpu as pltpu
from jax.experimental.pallas import tpu_sc as plsc

_INFO = plsc.get_sparse_core_info()
_NC, _NS = _INFO.num_cores, _INFO.num_subcores
_NW = _NC * _NS
_CHUNK = 128
_B = 4 * 8192
_PER_W = _B // _NW
_NCHUNK = _PER_W // _CHUNK
_NBUF = 7
_D = 128


_ROWS, _COLS = 4, 8192
_WPR = _NW // _ROWS


@functools.partial(
    pl.kernel,
    out_type=jax.ShapeDtypeStruct((_ROWS, _COLS, _D), jnp.float32),
    mesh=plsc.VectorSubcoreMesh(core_axis_name="c", subcore_axis_name="s"),
    scratch_types=[
        pltpu.VMEM((_PER_W,), jnp.int32),
        pltpu.VMEM((_NBUF, _CHUNK, _D), jnp.float32),
        pltpu.SemaphoreType.DMA,
        pltpu.SemaphoreType.DMA,
    ],
)
def _embed_gather(idx_hbm, table_hbm, out_hbm, idx_v, rows_v, gsem, ssem):
    wid = lax.axis_index("s") * _NC + lax.axis_index("c")
    r = wid // _WPR
    col0 = (wid % _WPR) * _PER_W
    pltpu.sync_copy(idx_hbm.at[r, pl.ds(col0, _PER_W)], idx_v)
    gathers = [None] * _NCHUNK
    stores = [None] * _NCHUNK
    for j in range(_NBUF):
        gathers[j] = pltpu.async_copy(
            table_hbm.at[idx_v.at[pl.ds(j * _CHUNK, _CHUNK)]], rows_v.at[j],
            gsem)
    for j in range(_NCHUNK):
        b = j % _NBUF
        gathers[j].wait()
        stores[j] = pltpu.async_copy(
            rows_v.at[b], out_hbm.at[r, pl.ds(col0 + j * _CHUNK, _CHUNK)],
            ssem)
        fj = j - 1 + _NBUF
        if 0 <= j - 1 and fj < _NCHUNK:
            stores[j - 1].wait()
            gathers[fj] = pltpu.async_copy(
                table_hbm.at[idx_v.at[pl.ds(fj * _CHUNK, _CHUNK)]],
                rows_v.at[(j - 1) % _NBUF], gsem)
    for j in range(_NCHUNK):
        if not (0 <= j <= _NCHUNK - _NBUF - 1):
            stores[j].wait()


def kernel(tokens, embedding):
    return _embed_gather(tokens.astype(jnp.int32), embedding)


# speedup vs baseline: 1.0061x; 1.0061x over previous
import functools

import jax
import jax.numpy as jnp
from jax import lax
from jax.experimental import pallas as pl
from jax.experimental.pallas import tpu as pltpu
from jax.experimental.pallas import tpu_sc as plsc

_INFO = plsc.get_sparse_core_info()
_NC, _NS = _INFO.num_cores, _INFO.num_subcores
_NW = _NC * _NS
_B = 4 * 8192
_PER_W = _B // _NW
_CHUNKS = (384, 384, 256)
_OFFS = (0, 384, 768)
_NCHUNK = len(_CHUNKS)
_NBUF = 2
_BUFROWS = max(_CHUNKS)
_D = 128


_ROWS, _COLS = 4, 8192
_WPR = _NW // _ROWS


@functools.partial(
    pl.kernel,
    out_type=jax.ShapeDtypeStruct((_ROWS, _COLS, _D), jnp.float32),
    mesh=plsc.VectorSubcoreMesh(core_axis_name="c", subcore_axis_name="s"),
    scratch_types=[
        pltpu.VMEM((_PER_W,), jnp.int32),
        pltpu.VMEM((_NBUF, _BUFROWS, _D), jnp.float32),
        pltpu.SemaphoreType.DMA,
        pltpu.SemaphoreType.DMA,
    ],
)
def _embed_gather(idx_hbm, table_hbm, out_hbm, idx_v, rows_v, gsem, ssem):
    wid = lax.axis_index("s") * _NC + lax.axis_index("c")
    r = wid // _WPR
    col0 = (wid % _WPR) * _PER_W

    def gather(j, b):
        return pltpu.async_copy(
            table_hbm.at[idx_v.at[pl.ds(_OFFS[j], _CHUNKS[j])]],
            rows_v.at[b, pl.ds(0, _CHUNKS[j])], gsem)

    def store(j, b):
        return pltpu.async_copy(
            rows_v.at[b, pl.ds(0, _CHUNKS[j])],
            out_hbm.at[r, pl.ds(col0 + _OFFS[j], _CHUNKS[j])], ssem)

    pltpu.sync_copy(idx_hbm.at[r, pl.ds(col0, _PER_W)], idx_v)
    gathers = [None] * _NCHUNK
    stores = [None] * _NCHUNK
    for j in range(_NBUF):
        gathers[j] = gather(j, j)
    for j in range(_NCHUNK):
        gathers[j].wait()
        stores[j] = store(j, j % _NBUF)
        fj = j - 1 + _NBUF
        if 0 <= j - 1 and fj < _NCHUNK:
            stores[j - 1].wait()
            gathers[fj] = gather(fj, (j - 1) % _NBUF)
    for j in range(_NCHUNK):
        if not (0 <= j <= _NCHUNK - _NBUF - 1):
            stores[j].wait()


def kernel(tokens, embedding):
    return _embed_gather(tokens.astype(jnp.int32), embedding)


# speedup vs baseline: 1.0169x; 1.0107x over previous
import functools

import jax
import jax.numpy as jnp
from jax import lax
from jax.experimental import pallas as pl
from jax.experimental.pallas import tpu as pltpu
from jax.experimental.pallas import tpu_sc as plsc

_INFO = plsc.get_sparse_core_info()
_NC, _NS = _INFO.num_cores, _INFO.num_subcores
_NW = _NC * _NS
_B = 4 * 8192
_PER_W = _B // _NW
_CHUNKS = (256, 256, 256, 256)
_OFFS = (0, 256, 512, 768)
_NCHUNK = len(_CHUNKS)
_NBUF = 3
_BUFROWS = max(_CHUNKS)
_D = 128


_ROWS, _COLS = 4, 8192
_WPR = _NW // _ROWS


@functools.partial(
    pl.kernel,
    out_type=jax.ShapeDtypeStruct((_ROWS, _COLS, _D), jnp.float32),
    mesh=plsc.VectorSubcoreMesh(core_axis_name="c", subcore_axis_name="s"),
    scratch_types=[
        pltpu.VMEM((_PER_W,), jnp.int32),
        pltpu.VMEM((_NBUF, _BUFROWS, _D), jnp.float32),
        pltpu.SemaphoreType.DMA,
        pltpu.SemaphoreType.DMA,
    ],
)
def _embed_gather(idx_hbm, table_hbm, out_hbm, idx_v, rows_v, gsem, ssem):
    wid = lax.axis_index("s") * _NC + lax.axis_index("c")
    r = wid // _WPR
    col0 = (wid % _WPR) * _PER_W

    def gather(j, b):
        return pltpu.async_copy(
            table_hbm.at[idx_v.at[pl.ds(_OFFS[j], _CHUNKS[j])]],
            rows_v.at[b, pl.ds(0, _CHUNKS[j])], gsem)

    def store(j, b):
        return pltpu.async_copy(
            rows_v.at[b, pl.ds(0, _CHUNKS[j])],
            out_hbm.at[r, pl.ds(col0 + _OFFS[j], _CHUNKS[j])], ssem)

    pltpu.sync_copy(idx_hbm.at[r, pl.ds(col0, _PER_W)], idx_v)
    gathers = [None] * _NCHUNK
    stores = [None] * _NCHUNK
    for j in range(_NBUF):
        gathers[j] = gather(j, j)
    for j in range(_NCHUNK):
        gathers[j].wait()
        stores[j] = store(j, j % _NBUF)
        fj = j + _NBUF
        if fj < _NCHUNK:
            stores[j].wait()
            gathers[fj] = gather(fj, j % _NBUF)
    for j in range(max(0, _NCHUNK - _NBUF), _NCHUNK):
        stores[j].wait()


def kernel(tokens, embedding):
    return _embed_gather(tokens.astype(jnp.int32), embedding)
